# baseline (speedup 1.0000x reference)
import functools

import jax
import jax.numpy as jnp
from jax import lax
from jax.experimental import pallas as pl
from jax.experimental.pallas import tpu as pltpu
from jax.experimental.pallas import tpu_sc as plsc

D_MODEL = 768
D_HID = 256
D_FF = 2 * D_HID
N_EXP = 64
T = 2048

NW = 32
BPW = T // NW
TM = 512


def _sc_gather_rows(table, idx):
    B = idx.shape[0]
    D = table.shape[1]
    mesh = plsc.VectorSubcoreMesh(core_axis_name="c", subcore_axis_name="s")

    @functools.partial(
        pl.kernel,
        mesh=mesh,
        out_type=jax.ShapeDtypeStruct((B, D), jnp.float32),
        scratch_types=[
            pltpu.VMEM((BPW,), jnp.int32),
            pltpu.VMEM((BPW, D), jnp.float32),
            pltpu.SemaphoreType.DMA,
        ],
    )
    def gather_k(table_hbm, idx_hbm, out_hbm, idx_v, rows_v, sem):
        wid = lax.axis_index("s") * 2 + lax.axis_index("c")
        base = wid * BPW
        pltpu.sync_copy(idx_hbm.at[pl.ds(base, BPW)], idx_v)
        pltpu.async_copy(table_hbm.at[idx_v], rows_v, sem).wait()
        pltpu.sync_copy(rows_v, out_hbm.at[pl.ds(base, BPW)])

    return gather_k(table, idx)


def _sc_dispatch(x, scale2d, order):
    mesh = plsc.VectorSubcoreMesh(core_axis_name="c", subcore_axis_name="s")

    @functools.partial(
        pl.kernel,
        mesh=mesh,
        out_type=[
            jax.ShapeDtypeStruct((T, D_MODEL), jnp.float32),
            jax.ShapeDtypeStruct((T, 128), jnp.float32),
        ],
        scratch_types=[
            pltpu.VMEM((BPW,), jnp.int32),
            pltpu.VMEM((BPW, D_MODEL), jnp.float32),
            pltpu.VMEM((BPW, 128), jnp.float32),
            pltpu.SemaphoreType.DMA,
            pltpu.SemaphoreType.DMA,
        ],
    )
    def dispatch_k(x_hbm, scale_hbm, ord_hbm, xs_hbm, ss_hbm,
                   idx_v, rows_v, srows_v, sem1, sem2):
        wid = lax.axis_index("s") * 2 + lax.axis_index("c")
        base = wid * BPW
        pltpu.sync_copy(ord_hbm.at[pl.ds(base, BPW)], idx_v)
        cp1 = pltpu.async_copy(x_hbm.at[idx_v], rows_v, sem1)
        cp2 = pltpu.async_copy(scale_hbm.at[idx_v], srows_v, sem2)
        cp1.wait()
        cp2.wait()
        pltpu.sync_copy(rows_v, xs_hbm.at[pl.ds(base, BPW)])
        pltpu.sync_copy(srows_v, ss_hbm.at[pl.ds(base, BPW)])

    return dispatch_k(x, scale2d, order)


def _tc_gmm_kernel(offs_ref, xs_ref, ss_ref, w1_ref, w2_ref, out_ref):
    e = pl.program_id(0)

    @pl.when(e == 0)
    def _init():
        out_ref[...] = jnp.zeros_like(out_ref)

    start = offs_ref[e]
    end = offs_ref[e + 1]
    t0 = start // TM
    t1 = (end + TM - 1) // TM
    w1 = w1_ref[0]
    w2 = w2_ref[0]

    def body(ti, carry):
        r0 = ti * TM
        rows = xs_ref[pl.ds(r0, TM), :]
        y = lax.dot_general(rows, w1, (((1,), (1,)), ((), ())),
                            preferred_element_type=jnp.float32)
        y1 = y[:, :D_HID]
        g = y[:, D_HID:]
        h = y1 * g * jax.nn.sigmoid(g)
        yo = lax.dot_general(h, w2, (((1,), (1,)), ((), ())),
                             preferred_element_type=jnp.float32)
        rid = r0 + lax.broadcasted_iota(jnp.int32, (TM, 1), 0)
        m = (rid >= start) & (rid < end)
        sc = ss_ref[pl.ds(r0, TM), 0:1]
        out_ref[pl.ds(r0, TM), :] += jnp.where(m, sc * yo, 0.0)
        return carry

    lax.fori_loop(t0, t1, body, 0)


def _tc_gmm(offsets, xs, ss, fc1_weights, fc2_weights):
    return pl.pallas_call(
        _tc_gmm_kernel,
        grid=(N_EXP,),
        in_specs=[
            pl.BlockSpec(memory_space=pltpu.SMEM),
            pl.BlockSpec((T, D_MODEL), lambda e: (0, 0)),
            pl.BlockSpec((T, 128), lambda e: (0, 0)),
            pl.BlockSpec((1, D_FF, D_MODEL), lambda e: (e, 0, 0)),
            pl.BlockSpec((1, D_MODEL, D_HID), lambda e: (e, 0, 0)),
        ],
        out_specs=pl.BlockSpec((T, D_MODEL), lambda e: (0, 0)),
        out_shape=jax.ShapeDtypeStruct((T, D_MODEL), jnp.float32),
        compiler_params=pltpu.CompilerParams(
            dimension_semantics=("arbitrary",)),
    )(offsets, xs, ss, fc1_weights, fc2_weights)


def kernel(x, weights, indices, fc1_weights, fc2_weights):
    e_ids = indices[:, 0].astype(jnp.int32)
    scale = weights[:, 0].astype(jnp.float32)
    order = jnp.argsort(e_ids).astype(jnp.int32)
    rank = jnp.argsort(order).astype(jnp.int32)
    counts = jnp.bincount(e_ids, length=N_EXP)
    offsets = jnp.concatenate(
        [jnp.zeros((1,), jnp.int32), jnp.cumsum(counts).astype(jnp.int32)])

    scale2d = jnp.broadcast_to(scale[:, None], (T, 128))
    xs, ss = _sc_dispatch(x, scale2d, order)
    out_sorted = _tc_gmm(offsets, xs, ss, fc1_weights, fc2_weights)
    return _sc_gather_rows(out_sorted, rank)


# speedup vs baseline: 1.1271x; 1.1271x over previous
import functools

import jax
import jax.numpy as jnp
from jax import lax
from jax.experimental import pallas as pl
from jax.experimental.pallas import tpu as pltpu
from jax.experimental.pallas import tpu_sc as plsc

D_MODEL = 768
D_HID = 256
D_FF = 2 * D_HID
N_EXP = 64
T = 2048

NW = 32
BPW = T // NW
TM = 256


def _sc_scatter_rows(table, idx):
    B = idx.shape[0]
    D = table.shape[1]
    mesh = plsc.VectorSubcoreMesh(core_axis_name="c", subcore_axis_name="s")

    @functools.partial(
        pl.kernel,
        mesh=mesh,
        out_type=jax.ShapeDtypeStruct((B, D), jnp.float32),
        scratch_types=[
            pltpu.VMEM((BPW,), jnp.int32),
            pltpu.VMEM((BPW, D), jnp.float32),
            pltpu.SemaphoreType.DMA,
        ],
    )
    def scatter_k(table_hbm, idx_hbm, out_hbm, idx_v, rows_v, sem):
        wid = lax.axis_index("s") * 2 + lax.axis_index("c")
        base = wid * BPW
        pltpu.sync_copy(idx_hbm.at[pl.ds(base, BPW)], idx_v)
        pltpu.sync_copy(table_hbm.at[pl.ds(base, BPW)], rows_v)
        pltpu.async_copy(rows_v, out_hbm.at[idx_v], sem).wait()

    return scatter_k(table, idx)


def _sc_dispatch(x, scale2d, order):
    mesh = plsc.VectorSubcoreMesh(core_axis_name="c", subcore_axis_name="s")

    @functools.partial(
        pl.kernel,
        mesh=mesh,
        out_type=[
            jax.ShapeDtypeStruct((T, D_MODEL), jnp.float32),
            jax.ShapeDtypeStruct((T, 128), jnp.float32),
        ],
        scratch_types=[
            pltpu.VMEM((BPW,), jnp.int32),
            pltpu.VMEM((BPW, D_MODEL), jnp.float32),
            pltpu.VMEM((BPW, 128), jnp.float32),
            pltpu.SemaphoreType.DMA,
            pltpu.SemaphoreType.DMA,
        ],
    )
    def dispatch_k(x_hbm, scale_hbm, ord_hbm, xs_hbm, ss_hbm,
                   idx_v, rows_v, srows_v, sem1, sem2):
        wid = lax.axis_index("s") * 2 + lax.axis_index("c")
        base = wid * BPW
        pltpu.sync_copy(ord_hbm.at[pl.ds(base, BPW)], idx_v)
        cp1 = pltpu.async_copy(x_hbm.at[idx_v], rows_v, sem1)
        cp2 = pltpu.async_copy(scale_hbm.at[idx_v], srows_v, sem2)
        cp1.wait()
        cp2.wait()
        pltpu.sync_copy(rows_v, xs_hbm.at[pl.ds(base, BPW)])
        pltpu.sync_copy(srows_v, ss_hbm.at[pl.ds(base, BPW)])

    return dispatch_k(x, scale2d, order)


def _tc_gmm_kernel(offs_ref, xs_ref, ss_ref, w1_ref, w2_ref, out_ref):
    e = pl.program_id(0)

    @pl.when(e == 0)
    def _init():
        out_ref[...] = jnp.zeros_like(out_ref)

    start = offs_ref[e]
    end = offs_ref[e + 1]
    t0 = start // TM
    t1 = (end + TM - 1) // TM
    w1 = w1_ref[0]
    w2 = w2_ref[0]

    def body(ti, carry):
        r0 = ti * TM
        rows = xs_ref[pl.ds(r0, TM), :]
        y = lax.dot_general(rows, w1, (((1,), (1,)), ((), ())),
                            preferred_element_type=jnp.float32)
        y1 = y[:, :D_HID]
        g = y[:, D_HID:]
        h = y1 * g * jax.nn.sigmoid(g)
        yo = lax.dot_general(h, w2, (((1,), (1,)), ((), ())),
                             preferred_element_type=jnp.float32)
        rid = r0 + lax.broadcasted_iota(jnp.int32, (TM, 1), 0)
        m = (rid >= start) & (rid < end)
        sc = ss_ref[pl.ds(r0, TM), 0:1]
        out_ref[pl.ds(r0, TM), :] += jnp.where(m, sc * yo, 0.0)
        return carry

    lax.fori_loop(t0, t1, body, 0)


def _tc_gmm(offsets, xs, ss, fc1_weights, fc2_weights):
    return pl.pallas_call(
        _tc_gmm_kernel,
        grid=(N_EXP,),
        in_specs=[
            pl.BlockSpec(memory_space=pltpu.SMEM),
            pl.BlockSpec((T, D_MODEL), lambda e: (0, 0)),
            pl.BlockSpec((T, 128), lambda e: (0, 0)),
            pl.BlockSpec((1, D_FF, D_MODEL), lambda e: (e, 0, 0)),
            pl.BlockSpec((1, D_MODEL, D_HID), lambda e: (e, 0, 0)),
        ],
        out_specs=pl.BlockSpec((T, D_MODEL), lambda e: (0, 0)),
        out_shape=jax.ShapeDtypeStruct((T, D_MODEL), jnp.float32),
        compiler_params=pltpu.CompilerParams(
            dimension_semantics=("arbitrary",)),
    )(offsets, xs, ss, fc1_weights, fc2_weights)


def kernel(x, weights, indices, fc1_weights, fc2_weights):
    e_ids = indices[:, 0].astype(jnp.int32)
    scale = weights[:, 0].astype(jnp.float32)
    order = jnp.argsort(e_ids).astype(jnp.int32)
    counts = jnp.sum(
        e_ids[:, None] == jnp.arange(N_EXP, dtype=jnp.int32)[None, :], axis=0)
    offsets = jnp.concatenate(
        [jnp.zeros((1,), jnp.int32), jnp.cumsum(counts).astype(jnp.int32)])

    scale2d = jnp.broadcast_to(scale[:, None], (T, 128))
    xs, ss = _sc_dispatch(x, scale2d, order)
    out_sorted = _tc_gmm(offsets, xs, ss, fc1_weights, fc2_weights)
    return _sc_scatter_rows(out_sorted, order)


# speedup vs baseline: 1.1306x; 1.0031x over previous
import functools

import jax
import jax.numpy as jnp
from jax import lax
from jax.experimental import pallas as pl
from jax.experimental.pallas import tpu as pltpu
from jax.experimental.pallas import tpu_sc as plsc

D_MODEL = 768
D_HID = 256
D_FF = 2 * D_HID
N_EXP = 64
T = 2048

NW = 32
BPW = T // NW
TM = 256


def _sc_scatter_rows(table, idx):
    B = idx.shape[0]
    D = table.shape[1]
    mesh = plsc.VectorSubcoreMesh(core_axis_name="c", subcore_axis_name="s")

    @functools.partial(
        pl.kernel,
        mesh=mesh,
        out_type=jax.ShapeDtypeStruct((B, D), jnp.float32),
        scratch_types=[
            pltpu.VMEM((BPW,), jnp.int32),
            pltpu.VMEM((BPW, D), jnp.float32),
            pltpu.SemaphoreType.DMA,
        ],
    )
    def scatter_k(table_hbm, idx_hbm, out_hbm, idx_v, rows_v, sem):
        wid = lax.axis_index("s") * 2 + lax.axis_index("c")
        base = wid * BPW
        pltpu.sync_copy(idx_hbm.at[pl.ds(base, BPW)], idx_v)
        pltpu.sync_copy(table_hbm.at[pl.ds(base, BPW)], rows_v)
        pltpu.async_copy(rows_v, out_hbm.at[idx_v], sem).wait()

    return scatter_k(table, idx)


def _sc_dispatch(x, scale2d, order):
    mesh = plsc.VectorSubcoreMesh(core_axis_name="c", subcore_axis_name="s")

    @functools.partial(
        pl.kernel,
        mesh=mesh,
        out_type=[
            jax.ShapeDtypeStruct((T, D_MODEL), jnp.float32),
            jax.ShapeDtypeStruct((T, 128), jnp.float32),
        ],
        scratch_types=[
            pltpu.VMEM((BPW,), jnp.int32),
            pltpu.VMEM((BPW, D_MODEL), jnp.float32),
            pltpu.VMEM((BPW, 128), jnp.float32),
            pltpu.SemaphoreType.DMA,
            pltpu.SemaphoreType.DMA,
        ],
    )
    def dispatch_k(x_hbm, scale_hbm, ord_hbm, xs_hbm, ss_hbm,
                   idx_v, rows_v, srows_v, sem1, sem2):
        wid = lax.axis_index("s") * 2 + lax.axis_index("c")
        base = wid * BPW
        pltpu.sync_copy(ord_hbm.at[pl.ds(base, BPW)], idx_v)
        cp1 = pltpu.async_copy(x_hbm.at[idx_v], rows_v, sem1)
        cp2 = pltpu.async_copy(scale_hbm.at[idx_v], srows_v, sem2)
        cp1.wait()
        cp2.wait()
        pltpu.sync_copy(rows_v, xs_hbm.at[pl.ds(base, BPW)])
        pltpu.sync_copy(srows_v, ss_hbm.at[pl.ds(base, BPW)])

    return dispatch_k(x, scale2d, order)


def _tc_gmm_kernel(offs_ref, xs_ref, ss_ref, w1_ref, w2_ref, out_ref):
    e = pl.program_id(0)

    @pl.when(e == 0)
    def _init():
        out_ref[...] = jnp.zeros_like(out_ref)

    start = offs_ref[e]
    end = offs_ref[e + 1]
    t0 = start // TM
    t1 = (end + TM - 1) // TM
    w1 = w1_ref[0]
    w2 = w2_ref[0]

    def body(ti, carry):
        r0 = ti * TM
        rows = xs_ref[pl.ds(r0, TM), :]
        y = lax.dot_general(rows, w1, (((1,), (1,)), ((), ())),
                            preferred_element_type=jnp.float32)
        y1 = y[:, :D_HID]
        g = y[:, D_HID:]
        rid = r0 + lax.broadcasted_iota(jnp.int32, (TM, 1), 0)
        m = (rid >= start) & (rid < end)
        sc = ss_ref[pl.ds(r0, TM), 0:1]
        msc = jnp.where(m, sc, 0.0)
        h = y1 * g * jax.nn.sigmoid(g) * msc
        yo = lax.dot_general(h, w2, (((1,), (1,)), ((), ())),
                             preferred_element_type=jnp.float32)
        out_ref[pl.ds(r0, TM), :] += yo
        return carry

    lax.fori_loop(t0, t1, body, 0)


def _tc_gmm(offsets, xs, ss, fc1_weights, fc2_weights):
    return pl.pallas_call(
        _tc_gmm_kernel,
        grid=(N_EXP,),
        in_specs=[
            pl.BlockSpec(memory_space=pltpu.SMEM),
            pl.BlockSpec((T, D_MODEL), lambda e: (0, 0)),
            pl.BlockSpec((T, 128), lambda e: (0, 0)),
            pl.BlockSpec((1, D_FF, D_MODEL), lambda e: (e, 0, 0)),
            pl.BlockSpec((1, D_MODEL, D_HID), lambda e: (e, 0, 0)),
        ],
        out_specs=pl.BlockSpec((T, D_MODEL), lambda e: (0, 0)),
        out_shape=jax.ShapeDtypeStruct((T, D_MODEL), jnp.float32),
        compiler_params=pltpu.CompilerParams(
            dimension_semantics=("arbitrary",)),
    )(offsets, xs, ss, fc1_weights, fc2_weights)


def kernel(x, weights, indices, fc1_weights, fc2_weights):
    e_ids = indices[:, 0].astype(jnp.int32)
    scale = weights[:, 0].astype(jnp.float32)
    order = jnp.argsort(e_ids).astype(jnp.int32)
    counts = jnp.sum(
        e_ids[:, None] == jnp.arange(N_EXP, dtype=jnp.int32)[None, :], axis=0)
    offsets = jnp.concatenate(
        [jnp.zeros((1,), jnp.int32), jnp.cumsum(counts).astype(jnp.int32)])

    scale2d = jnp.broadcast_to(scale[:, None], (T, 128))
    xs, ss = _sc_dispatch(x, scale2d, order)
    out_sorted = _tc_gmm(offsets, xs, ss, fc1_weights, fc2_weights)
    return _sc_scatter_rows(out_sorted, order)
